# baseline (speedup 1.0000x reference)
import functools

import jax
import jax.numpy as jnp
from jax import lax
from jax.experimental import pallas as pl
from jax.experimental.pallas import tpu as pltpu
from jax.experimental.pallas import tpu_sc as plsc

EPS = 1e-5
CH = 128
NC = 2
NS = 16
D = 128


def _drain(acc, stage, out_ref, s, n_nodes):
  bpt = (n_nodes // NS) // 8 * 8
  rem = n_nodes - NS * bpt
  for k in range(0, bpt, CH):
    sz = min(CH, bpt - k)
    dyn = s * bpt + k
    pltpu.sync_copy(acc.at[pl.ds(dyn, sz)], stage.at[pl.ds(0, sz)])
    pltpu.sync_copy(stage.at[pl.ds(0, sz)], out_ref.at[pl.ds(dyn, sz)])
  if rem:
    @pl.when(s == NS - 1)
    def _():
      off = NS * bpt
      pltpu.sync_copy(acc.at[pl.ds(off, rem)], stage.at[pl.ds(0, rem)])
      pltpu.sync_copy(stage.at[pl.ds(0, rem)], out_ref.at[pl.ds(off, rem)])


def _hist_kernel(n_nodes, n_chunks):
  acc_rows = ((n_nodes + 16 + NS * 8 - 1) // (NS * 8)) * (NS * 8)
  rpt = acc_rows // NS
  cpt = n_chunks // (NC * NS)
  mesh = plsc.VectorSubcoreMesh(core_axis_name="c", subcore_axis_name="s")

  depth = 4
  @functools.partial(
      pl.kernel,
      out_type=jax.ShapeDtypeStruct((NC, n_nodes, D), jnp.float32),
      mesh=mesh,
      scratch_types=[
          pltpu.VMEM((cpt, 1, CH), jnp.int32),
          pltpu.VMEM((CH, D), jnp.float32),
          pltpu.VMEM_SHARED((acc_rows, D), jnp.float32),
          pltpu.SemaphoreType.DMA,
          pltpu.SemaphoreType.DMA,
      ],
  )
  def hist(zeros_hbm, ones_hbm, dst_hbm, out_hbm, idx_d, rows, acc,
           ssem, hsem):
    c = lax.axis_index("c")
    s = lax.axis_index("s")
    wid = s * NC + c
    ic = pltpu.async_copy(dst_hbm.at[pl.ds(wid * cpt, cpt)], idx_d, hsem)
    pltpu.sync_copy(zeros_hbm, rows)
    for k in range(0, rpt, CH):
      sz = min(CH, rpt - k)
      pltpu.sync_copy(rows.at[pl.ds(0, sz)],
                      acc.at[pl.ds(s * rpt + k, sz)])
    pltpu.sync_copy(ones_hbm, rows)
    ic.wait()
    plsc.subcore_barrier()

    def body(j, _):
      pltpu.async_copy(rows, acc.at[idx_d.at[j, 0]], ssem, add=True)
      @pl.when(j >= depth)
      def _():
        pltpu.make_async_copy(zeros_hbm, rows, ssem).wait()
      return 0
    lax.fori_loop(0, cpt, body, 0)
    for _ in range(depth):
      pltpu.make_async_copy(zeros_hbm, rows, ssem).wait()
    plsc.subcore_barrier()
    _drain(acc, rows, out_hbm.at[c], s, n_nodes)

  return hist


NBUF = 4


def _agg_kernel(n_nodes, n_chunks, cpt0=None):
  acc_rows = ((n_nodes + 16 + NS * 8 - 1) // (NS * 8)) * (NS * 8)
  rpt = acc_rows // NS
  cps = n_chunks // NS
  if cpt0 is None:
    cpt0 = (cps // 2) // 2 * 2
  cpt1 = cps - cpt0
  assert cpt0 % 2 == 0 and cpt1 % 2 == 0
  mesh = plsc.VectorSubcoreMesh(core_axis_name="c", subcore_axis_name="s")

  @functools.partial(
      pl.kernel,
      out_type=jax.ShapeDtypeStruct((NC, n_nodes, D), jnp.float32),
      mesh=mesh,
      scratch_types=[
          pltpu.VMEM((2, CH), jnp.int32),
          pltpu.VMEM((2, CH), jnp.int32),
          pltpu.VMEM((2, CH, D), jnp.float32),
          pltpu.VMEM_SHARED((acc_rows, D), jnp.float32),
          [pltpu.SemaphoreType.DMA] * 2,
          [pltpu.SemaphoreType.DMA] * 2,
      ],
  )
  def agg(table_hbm, zeros_hbm, src_hbm, dst_hbm, out_hbm,
          idx_s, idx_d, rows, acc, gsem, ssem):
    c = lax.axis_index("c")
    s = lax.axis_index("s")
    cpt = jnp.where(c == 0, cpt0, cpt1)
    base = jnp.where(c == 0, s * cpt0, NS * cpt0 + s * cpt1)
    pltpu.sync_copy(zeros_hbm, rows.at[0])
    for k in range(0, rpt, CH):
      sz = min(CH, rpt - k)
      pltpu.sync_copy(rows.at[0].at[pl.ds(0, sz)],
                      acc.at[pl.ds(s * rpt + k, sz)])
    plsc.subcore_barrier()
    pltpu.sync_copy(src_hbm.at[base, 0], idx_s.at[0])
    pltpu.sync_copy(dst_hbm.at[base, 0], idx_d.at[0])
    pltpu.async_copy(table_hbm.at[idx_s.at[0]], rows.at[0], gsem[0])

    def half(i, p):
      j = 2 * i + p

      def wait_prev_scatter():
        pltpu.make_async_copy(zeros_hbm, rows.at[1 - p], ssem[1 - p]).wait()
      if p == 0:
        @pl.when(i > 0)
        def _():
          wait_prev_scatter()
      else:
        wait_prev_scatter()
      @pl.when(j + 1 < cpt)
      def _():
        g = base + j + 1
        pltpu.sync_copy(src_hbm.at[g, 0], idx_s.at[1 - p])
        pltpu.sync_copy(dst_hbm.at[g, 0], idx_d.at[1 - p])
        pltpu.async_copy(table_hbm.at[idx_s.at[1 - p]], rows.at[1 - p],
                         gsem[1 - p])
      pltpu.make_async_copy(zeros_hbm, rows.at[p], gsem[p]).wait()
      pltpu.async_copy(rows.at[p], acc.at[idx_d.at[p]], ssem[p], add=True)

    def body(i, _):
      half(i, 0)
      half(i, 1)
      return 0
    lax.fori_loop(0, cpt // 2, body, 0)
    pltpu.make_async_copy(zeros_hbm, rows.at[1], ssem[1]).wait()
    plsc.subcore_barrier()
    _drain(acc, rows.at[0], out_hbm.at[c], s, n_nodes)

  return agg


def _mm(x, w0t):
  n, d_in = x.shape
  bm = 1000
  grid = (n // bm,)

  def body(x_ref, w_ref, t_ref):
    t_ref[...] = jnp.dot(x_ref[...], w_ref[...],
                         preferred_element_type=jnp.float32)

  return pl.pallas_call(
      body,
      grid=grid,
      in_specs=[
          pl.BlockSpec((bm, d_in), lambda i: (i, 0)),
          pl.BlockSpec((d_in, D), lambda i: (0, 0)),
      ],
      out_specs=pl.BlockSpec((bm, D), lambda i: (i, 0)),
      out_shape=jax.ShapeDtypeStruct((n, D), jnp.float32),
  )(x, w0t)


def _scale(degA, degB, mm):
  n = mm.shape[0]
  bm = 1000
  grid = (n // bm,)

  def body(da_ref, db_ref, mm_ref, t_ref, dis_ref):
    deg = da_ref[:, :1] + db_ref[:, :1]
    dis = jnp.where(deg > 0, lax.rsqrt(deg), 0.0)
    t_ref[...] = dis * mm_ref[...]
    dis_ref[...] = dis

  return pl.pallas_call(
      body,
      grid=grid,
      in_specs=[
          pl.BlockSpec((bm, D), lambda i: (i, 0)),
          pl.BlockSpec((bm, D), lambda i: (i, 0)),
          pl.BlockSpec((bm, D), lambda i: (i, 0)),
      ],
      out_specs=[
          pl.BlockSpec((bm, D), lambda i: (i, 0)),
          pl.BlockSpec((bm, 1), lambda i: (i, 0)),
      ],
      out_shape=[
          jax.ShapeDtypeStruct((n, D), jnp.float32),
          jax.ShapeDtypeStruct((n, 1), jnp.float32),
      ],
  )(degA, degB, mm)


def _combine_stats(accA, accB, dis):
  n = accA.shape[0]
  bm = 1000
  grid = (n // bm,)

  def body(a_ref, b_ref, dis_ref, u_ref, st_ref):
    i = pl.program_id(0)
    u = dis_ref[...] * (a_ref[...] + b_ref[...])
    u_ref[...] = u
    @pl.when(i == 0)
    def _():
      st_ref[...] = jnp.zeros((2, D), jnp.float32)
    st_ref[0:1, :] += jnp.sum(u, axis=0, keepdims=True)
    st_ref[1:2, :] += jnp.sum(u * u, axis=0, keepdims=True)

  return pl.pallas_call(
      body,
      grid=grid,
      in_specs=[
          pl.BlockSpec((bm, D), lambda i: (i, 0)),
          pl.BlockSpec((bm, D), lambda i: (i, 0)),
          pl.BlockSpec((bm, 1), lambda i: (i, 0)),
      ],
      out_specs=[
          pl.BlockSpec((bm, D), lambda i: (i, 0)),
          pl.BlockSpec((2, D), lambda i: (0, 0)),
      ],
      out_shape=[
          jax.ShapeDtypeStruct((n, D), jnp.float32),
          jax.ShapeDtypeStruct((2, D), jnp.float32),
      ],
  )(accA, accB, dis)


def _bn_relu_mm_scale(u, st, dis, w1t):
  n = u.shape[0]
  bm = 1000
  grid = (n // bm,)

  def body(u_ref, st_ref, dis_ref, w_ref, t_ref):
    mean = st_ref[0:1, :] / n
    var = st_ref[1:2, :] / n - mean * mean
    h = jnp.maximum((u_ref[...] - mean) * lax.rsqrt(var + EPS), 0.0)
    t_ref[...] = dis_ref[...] * jnp.dot(h, w_ref[...],
                                        preferred_element_type=jnp.float32)

  return pl.pallas_call(
      body,
      grid=grid,
      in_specs=[
          pl.BlockSpec((bm, D), lambda i: (i, 0)),
          pl.BlockSpec((2, D), lambda i: (0, 0)),
          pl.BlockSpec((bm, 1), lambda i: (i, 0)),
          pl.BlockSpec((D, D), lambda i: (0, 0)),
      ],
      out_specs=pl.BlockSpec((bm, D), lambda i: (i, 0)),
      out_shape=jax.ShapeDtypeStruct((n, D), jnp.float32),
  )(u, st, dis, w1t)


def _combine_bias(accA, accB, dis, b1):
  n = accA.shape[0]
  bm = 1000
  grid = (n // bm,)

  def body(a_ref, b_ref, dis_ref, b1_ref, o_ref):
    o_ref[...] = dis_ref[...] * (a_ref[...] + b_ref[...]) + b1_ref[...]

  return pl.pallas_call(
      body,
      grid=grid,
      in_specs=[
          pl.BlockSpec((bm, D), lambda i: (i, 0)),
          pl.BlockSpec((bm, D), lambda i: (i, 0)),
          pl.BlockSpec((bm, 1), lambda i: (i, 0)),
          pl.BlockSpec((1, D), lambda i: (0, 0)),
      ],
      out_specs=pl.BlockSpec((bm, D), lambda i: (i, 0)),
      out_shape=jax.ShapeDtypeStruct((n, D), jnp.float32),
  )(accA, accB, dis, b1)


def kernel(x, edge_index, W0, b0, W1, b1):
  n, _ = x.shape
  e = edge_index.shape[1]
  src = edge_index[0]
  dst = edge_index[1]

  grp = NC * NS
  n_chunks = ((e + CH - 1) // CH + grp - 1) // grp * grp
  e_pad = n_chunks * CH
  npad = e_pad - e
  acc_rows = ((n + 16 + NS * 8 - 1) // (NS * 8)) * (NS * 8)
  pad_dst = n + jnp.arange(npad, dtype=jnp.int32) % (acc_rows - n)
  src_p = jnp.concatenate(
      [src, jnp.zeros((npad,), jnp.int32)]).reshape(-1, 1, CH)
  dst_p = jnp.concatenate([dst, pad_dst]).reshape(-1, 1, CH)

  ones = jnp.ones((CH, D), jnp.float32)
  zeros = jnp.zeros((CH, D), jnp.float32)

  deg = _hist_kernel(n, n_chunks)(zeros, ones, dst_p)
  mm = _mm(x, W0.T)
  t0, dis = _scale(deg[0], deg[1], mm)
  agg = _agg_kernel(n, n_chunks, cpt0=102)
  acc0 = agg(t0, zeros, src_p, dst_p)
  u, st = _combine_stats(acc0[0], acc0[1], dis)
  t1 = _bn_relu_mm_scale(u, st, dis, W1.T)
  acc1 = agg(t1, zeros, src_p, dst_p)
  return _combine_bias(acc1[0], acc1[1], dis, b1.reshape(1, D))


# speedup vs baseline: 1.0016x; 1.0016x over previous
import functools

import jax
import jax.numpy as jnp
from jax import lax
from jax.experimental import pallas as pl
from jax.experimental.pallas import tpu as pltpu
from jax.experimental.pallas import tpu_sc as plsc

EPS = 1e-5
CH = 128
NC = 2
NS = 16
D = 128


def _drain(acc, stage, out_ref, s, n_nodes):
  bpt = (n_nodes // NS) // 8 * 8
  rem = n_nodes - NS * bpt
  for k in range(0, bpt, CH):
    sz = min(CH, bpt - k)
    dyn = s * bpt + k
    pltpu.sync_copy(acc.at[pl.ds(dyn, sz)], stage.at[pl.ds(0, sz)])
    pltpu.sync_copy(stage.at[pl.ds(0, sz)], out_ref.at[pl.ds(dyn, sz)])
  if rem:
    @pl.when(s == NS - 1)
    def _():
      off = NS * bpt
      pltpu.sync_copy(acc.at[pl.ds(off, rem)], stage.at[pl.ds(0, rem)])
      pltpu.sync_copy(stage.at[pl.ds(0, rem)], out_ref.at[pl.ds(off, rem)])


def _hist_kernel(n_nodes, n_chunks):
  acc_rows = ((n_nodes + 16 + NS * 8 - 1) // (NS * 8)) * (NS * 8)
  rpt = acc_rows // NS
  cpt = n_chunks // (NC * NS)
  mesh = plsc.VectorSubcoreMesh(core_axis_name="c", subcore_axis_name="s")

  depth = 4
  @functools.partial(
      pl.kernel,
      out_type=jax.ShapeDtypeStruct((NC, n_nodes, D), jnp.float32),
      mesh=mesh,
      scratch_types=[
          pltpu.VMEM((cpt, 1, CH), jnp.int32),
          pltpu.VMEM((CH, D), jnp.float32),
          pltpu.VMEM_SHARED((acc_rows, D), jnp.float32),
          pltpu.SemaphoreType.DMA,
          pltpu.SemaphoreType.DMA,
      ],
  )
  def hist(zeros_hbm, ones_hbm, dst_hbm, out_hbm, idx_d, rows, acc,
           ssem, hsem):
    c = lax.axis_index("c")
    s = lax.axis_index("s")
    wid = s * NC + c
    ic = pltpu.async_copy(dst_hbm.at[pl.ds(wid * cpt, cpt)], idx_d, hsem)
    pltpu.sync_copy(zeros_hbm, rows)
    for k in range(0, rpt, CH):
      sz = min(CH, rpt - k)
      pltpu.sync_copy(rows.at[pl.ds(0, sz)],
                      acc.at[pl.ds(s * rpt + k, sz)])
    pltpu.sync_copy(ones_hbm, rows)
    ic.wait()
    plsc.subcore_barrier()

    def body(j, _):
      pltpu.async_copy(rows, acc.at[idx_d.at[j, 0]], ssem, add=True)
      @pl.when(j >= depth)
      def _():
        pltpu.make_async_copy(zeros_hbm, rows, ssem).wait()
      return 0
    lax.fori_loop(0, cpt, body, 0)
    for _ in range(depth):
      pltpu.make_async_copy(zeros_hbm, rows, ssem).wait()
    plsc.subcore_barrier()
    _drain(acc, rows, out_hbm.at[c], s, n_nodes)

  return hist


NBUF = 4


def _agg_kernel(n_nodes, n_chunks, cpt0=None):
  acc_rows = ((n_nodes + 16 + NS * 8 - 1) // (NS * 8)) * (NS * 8)
  rpt = acc_rows // NS
  cps = n_chunks // NS
  if cpt0 is None:
    cpt0 = (cps // 2) // 2 * 2
  cpt1 = cps - cpt0
  assert cpt0 % 2 == 0 and cpt1 % 2 == 0
  mesh = plsc.VectorSubcoreMesh(core_axis_name="c", subcore_axis_name="s")

  @functools.partial(
      pl.kernel,
      out_type=jax.ShapeDtypeStruct((NC, n_nodes, D), jnp.float32),
      mesh=mesh,
      scratch_types=[
          pltpu.VMEM((2, CH), jnp.int32),
          pltpu.VMEM((2, CH), jnp.int32),
          pltpu.VMEM((2, CH, D), jnp.float32),
          pltpu.VMEM_SHARED((acc_rows, D), jnp.float32),
          [pltpu.SemaphoreType.DMA] * 2,
      ],
  )
  def agg(table_hbm, zeros_hbm, src_hbm, dst_hbm, out_hbm,
          idx_s, idx_d, rows, acc, gsem):
    c = lax.axis_index("c")
    s = lax.axis_index("s")
    cpt = jnp.where(c == 0, cpt0, cpt1)
    base = jnp.where(c == 0, s * cpt0, NS * cpt0 + s * cpt1)
    pltpu.sync_copy(zeros_hbm, rows.at[0])
    for k in range(0, rpt, CH):
      sz = min(CH, rpt - k)
      pltpu.sync_copy(rows.at[0].at[pl.ds(0, sz)],
                      acc.at[pl.ds(s * rpt + k, sz)])
    plsc.subcore_barrier()
    pltpu.sync_copy(src_hbm.at[base, 0], idx_s.at[0])
    pltpu.sync_copy(dst_hbm.at[base, 0], idx_d.at[0])
    pltpu.async_copy(table_hbm.at[idx_s.at[0]], rows.at[0], gsem[0])

    def half(i, p):
      j = 2 * i + p
      @pl.when(j + 1 < cpt)
      def _():
        g = base + j + 1
        pltpu.sync_copy(src_hbm.at[g, 0], idx_s.at[1 - p])
        pltpu.sync_copy(dst_hbm.at[g, 0], idx_d.at[1 - p])
        pltpu.async_copy(table_hbm.at[idx_s.at[1 - p]], rows.at[1 - p],
                         gsem[1 - p])
      pltpu.make_async_copy(zeros_hbm, rows.at[p], gsem[p]).wait()
      pltpu.sync_copy(rows.at[p], acc.at[idx_d.at[p]], add=True)

    def body(i, _):
      half(i, 0)
      half(i, 1)
      return 0
    lax.fori_loop(0, cpt // 2, body, 0)
    plsc.subcore_barrier()
    _drain(acc, rows.at[0], out_hbm.at[c], s, n_nodes)

  return agg


def _mm(x, w0t):
  n, d_in = x.shape
  bm = 1000
  grid = (n // bm,)

  def body(x_ref, w_ref, t_ref):
    t_ref[...] = jnp.dot(x_ref[...], w_ref[...],
                         preferred_element_type=jnp.float32)

  return pl.pallas_call(
      body,
      grid=grid,
      in_specs=[
          pl.BlockSpec((bm, d_in), lambda i: (i, 0)),
          pl.BlockSpec((d_in, D), lambda i: (0, 0)),
      ],
      out_specs=pl.BlockSpec((bm, D), lambda i: (i, 0)),
      out_shape=jax.ShapeDtypeStruct((n, D), jnp.float32),
  )(x, w0t)


def _scale(degA, degB, mm):
  n = mm.shape[0]
  bm = 1000
  grid = (n // bm,)

  def body(da_ref, db_ref, mm_ref, t_ref, dis_ref):
    deg = da_ref[:, :1] + db_ref[:, :1]
    dis = jnp.where(deg > 0, lax.rsqrt(deg), 0.0)
    t_ref[...] = dis * mm_ref[...]
    dis_ref[...] = dis

  return pl.pallas_call(
      body,
      grid=grid,
      in_specs=[
          pl.BlockSpec((bm, D), lambda i: (i, 0)),
          pl.BlockSpec((bm, D), lambda i: (i, 0)),
          pl.BlockSpec((bm, D), lambda i: (i, 0)),
      ],
      out_specs=[
          pl.BlockSpec((bm, D), lambda i: (i, 0)),
          pl.BlockSpec((bm, 1), lambda i: (i, 0)),
      ],
      out_shape=[
          jax.ShapeDtypeStruct((n, D), jnp.float32),
          jax.ShapeDtypeStruct((n, 1), jnp.float32),
      ],
  )(degA, degB, mm)


def _combine_stats(accA, accB, dis):
  n = accA.shape[0]
  bm = 1000
  grid = (n // bm,)

  def body(a_ref, b_ref, dis_ref, u_ref, st_ref):
    i = pl.program_id(0)
    u = dis_ref[...] * (a_ref[...] + b_ref[...])
    u_ref[...] = u
    @pl.when(i == 0)
    def _():
      st_ref[...] = jnp.zeros((2, D), jnp.float32)
    st_ref[0:1, :] += jnp.sum(u, axis=0, keepdims=True)
    st_ref[1:2, :] += jnp.sum(u * u, axis=0, keepdims=True)

  return pl.pallas_call(
      body,
      grid=grid,
      in_specs=[
          pl.BlockSpec((bm, D), lambda i: (i, 0)),
          pl.BlockSpec((bm, D), lambda i: (i, 0)),
          pl.BlockSpec((bm, 1), lambda i: (i, 0)),
      ],
      out_specs=[
          pl.BlockSpec((bm, D), lambda i: (i, 0)),
          pl.BlockSpec((2, D), lambda i: (0, 0)),
      ],
      out_shape=[
          jax.ShapeDtypeStruct((n, D), jnp.float32),
          jax.ShapeDtypeStruct((2, D), jnp.float32),
      ],
  )(accA, accB, dis)


def _bn_relu_mm_scale(u, st, dis, w1t):
  n = u.shape[0]
  bm = 1000
  grid = (n // bm,)

  def body(u_ref, st_ref, dis_ref, w_ref, t_ref):
    mean = st_ref[0:1, :] / n
    var = st_ref[1:2, :] / n - mean * mean
    h = jnp.maximum((u_ref[...] - mean) * lax.rsqrt(var + EPS), 0.0)
    t_ref[...] = dis_ref[...] * jnp.dot(h, w_ref[...],
                                        preferred_element_type=jnp.float32)

  return pl.pallas_call(
      body,
      grid=grid,
      in_specs=[
          pl.BlockSpec((bm, D), lambda i: (i, 0)),
          pl.BlockSpec((2, D), lambda i: (0, 0)),
          pl.BlockSpec((bm, 1), lambda i: (i, 0)),
          pl.BlockSpec((D, D), lambda i: (0, 0)),
      ],
      out_specs=pl.BlockSpec((bm, D), lambda i: (i, 0)),
      out_shape=jax.ShapeDtypeStruct((n, D), jnp.float32),
  )(u, st, dis, w1t)


def _combine_bias(accA, accB, dis, b1):
  n = accA.shape[0]
  bm = 1000
  grid = (n // bm,)

  def body(a_ref, b_ref, dis_ref, b1_ref, o_ref):
    o_ref[...] = dis_ref[...] * (a_ref[...] + b_ref[...]) + b1_ref[...]

  return pl.pallas_call(
      body,
      grid=grid,
      in_specs=[
          pl.BlockSpec((bm, D), lambda i: (i, 0)),
          pl.BlockSpec((bm, D), lambda i: (i, 0)),
          pl.BlockSpec((bm, 1), lambda i: (i, 0)),
          pl.BlockSpec((1, D), lambda i: (0, 0)),
      ],
      out_specs=pl.BlockSpec((bm, D), lambda i: (i, 0)),
      out_shape=jax.ShapeDtypeStruct((n, D), jnp.float32),
  )(accA, accB, dis, b1)


def kernel(x, edge_index, W0, b0, W1, b1):
  n, _ = x.shape
  e = edge_index.shape[1]
  src = edge_index[0]
  dst = edge_index[1]

  grp = NC * NS
  n_chunks = ((e + CH - 1) // CH + grp - 1) // grp * grp
  e_pad = n_chunks * CH
  npad = e_pad - e
  acc_rows = ((n + 16 + NS * 8 - 1) // (NS * 8)) * (NS * 8)
  pad_dst = n + jnp.arange(npad, dtype=jnp.int32) % (acc_rows - n)
  src_p = jnp.concatenate(
      [src, jnp.zeros((npad,), jnp.int32)]).reshape(-1, 1, CH)
  dst_p = jnp.concatenate([dst, pad_dst]).reshape(-1, 1, CH)

  ones = jnp.ones((CH, D), jnp.float32)
  zeros = jnp.zeros((CH, D), jnp.float32)

  deg = _hist_kernel(n, n_chunks)(zeros, ones, dst_p)
  mm = _mm(x, W0.T)
  t0, dis = _scale(deg[0], deg[1], mm)
  agg = _agg_kernel(n, n_chunks, cpt0=102)
  acc0 = agg(t0, zeros, src_p, dst_p)
  u, st = _combine_stats(acc0[0], acc0[1], dis)
  t1 = _bn_relu_mm_scale(u, st, dis, W1.T)
  acc1 = agg(t1, zeros, src_p, dst_p)
  return _combine_bias(acc1[0], acc1[1], dis, b1.reshape(1, D))
